# baseline (speedup 1.0000x reference)
import functools

import jax
import jax.numpy as jnp
from jax import lax
from jax.experimental import pallas as pl
from jax.experimental.pallas import tpu as pltpu
from jax.experimental.pallas import tpu_sc as plsc

N = 10000
E = 320000
D = 128
NP = 10240
TRASH = 10200
EP = 327680
NBT = EP // 128
ROWS_PER_W = NP // 16
BLK = 1024

_mesh = plsc.VectorSubcoreMesh(core_axis_name="c", subcore_axis_name="s")


@functools.partial(
    pl.kernel,
    mesh=_mesh,
    out_type=jax.ShapeDtypeStruct((2 * NP,), jnp.float32),
    scratch_types=[
        pltpu.VMEM((NBT // 32, 128), jnp.int32),
        pltpu.VMEM((128,), jnp.float32),
        pltpu.VMEM((ROWS_PER_W,), jnp.float32),
        pltpu.VMEM_SHARED((2 * NP,), jnp.float32),
    ],
)
def _sc_degree(dst_hbm, ones_hbm, zrow_hbm, out_hbm, dst_v, ones_v, zv, deg_sp):
    c = lax.axis_index("c")
    s = lax.axis_index("s")
    w = c * 16 + s
    nb = NBT // 32
    pltpu.sync_copy(dst_hbm.at[pl.ds(w * nb, nb)], dst_v)
    pltpu.sync_copy(ones_hbm, ones_v)
    pltpu.sync_copy(zrow_hbm.at[pl.ds(0, ROWS_PER_W)], zv)
    pltpu.sync_copy(zv, deg_sp.at[pl.ds(s * ROWS_PER_W, ROWS_PER_W)])
    plsc.subcore_barrier()

    def body(i, carry):
        pltpu.sync_copy(ones_v, deg_sp.at[dst_v.at[i]], add=True)
        return carry

    lax.fori_loop(0, nb, body, 0)
    plsc.subcore_barrier()
    pltpu.sync_copy(deg_sp.at[pl.ds(s * ROWS_PER_W, ROWS_PER_W)], zv)
    pltpu.sync_copy(zv, out_hbm.at[pl.ds(c * NP + s * ROWS_PER_W, ROWS_PER_W)])


DQ = D // 2


@functools.partial(
    pl.kernel,
    mesh=_mesh,
    out_type=jax.ShapeDtypeStruct((4, NP, DQ), jnp.float32),
    scratch_types=[
        pltpu.VMEM((NBT // 32, 128), jnp.int32),
        pltpu.VMEM((NBT // 32, 128), jnp.int32),
        pltpu.VMEM((128, DQ), jnp.float32),
        pltpu.VMEM((128, DQ), jnp.float32),
        pltpu.VMEM_SHARED((2 * NP, DQ), jnp.float32),
        pltpu.SemaphoreType.DMA,
        pltpu.SemaphoreType.DMA,
        pltpu.SemaphoreType.DMA,
        pltpu.SemaphoreType.DMA,
    ],
    compiler_params=pltpu.CompilerParams(use_tc_tiling_on_sc=False),
)
def _sc_pass(gpc_hbm, src4_hbm, dst_hbm, zrows_hbm, out_hbm,
             src_v, dst_v, rows_a, rows_b, acc_sp,
             gsem_a, gsem_b, ssem_a, ssem_b):
    c = lax.axis_index("c")
    s = lax.axis_index("s")
    nb = NBT // 16
    nh = nb // 2

    def gather(i, buf, sem):
        pltpu.async_copy(gpc_hbm.at[src_v.at[i]], buf, sem)

    def wait_gather(i, buf, sem):
        pltpu.make_async_copy(gpc_hbm.at[src_v.at[i]], buf, sem).wait()

    def scat(i, buf, sem):
        pltpu.async_copy(buf, acc_sp.at[dst_v.at[i]], sem, add=True)

    def wait_scat(i, buf, sem):
        pltpu.make_async_copy(buf, acc_sp.at[dst_v.at[i]], sem).wait()

    def body(j, carry):
        b0 = 2 * j
        wait_gather(b0, rows_a, gsem_a)
        scat(b0, rows_a, ssem_a)
        wait_gather(b0 + 1, rows_b, gsem_b)
        scat(b0 + 1, rows_b, ssem_b)
        wait_scat(b0, rows_a, ssem_a)

        @pl.when(b0 + 2 < nh)
        def _():
            gather(b0 + 2, rows_a, gsem_a)

        wait_scat(b0 + 1, rows_b, ssem_b)

        @pl.when(b0 + 3 < nh)
        def _():
            gather(b0 + 3, rows_b, gsem_b)

        return carry

    for f in range(2):
        q = 2 * c + f
        pltpu.sync_copy(zrows_hbm,
                        acc_sp.at[pl.ds(s * ROWS_PER_W, ROWS_PER_W)])
        plsc.subcore_barrier()
        for h in range(2):
            pltpu.sync_copy(src4_hbm.at[q, pl.ds(s * nb + h * nh, nh)], src_v)
            pltpu.sync_copy(dst_hbm.at[pl.ds(s * nb + h * nh, nh)], dst_v)
            gather(0, rows_a, gsem_a)
            gather(1, rows_b, gsem_b)
            lax.fori_loop(0, nh // 2, body, 0)
        plsc.subcore_barrier()
        pltpu.sync_copy(acc_sp.at[pl.ds(s * ROWS_PER_W, ROWS_PER_W)],
                        out_hbm.at[q, pl.ds(s * ROWS_PER_W, ROWS_PER_W)])


def _tc1_body(xp_ref, w_ref, b_ref, p0_ref, p1_ref,
              gpc_ref, sc_ref, dinv_ref):
    deg = 1.0 + p0_ref[0, :] + p1_ref[0, :]
    dinv = lax.rsqrt(deg)
    hw = jnp.dot(xp_ref[...], w_ref[...], preferred_element_type=jnp.float32)
    gp = dinv[:, None] * hw
    for q in range(4):
        gpc_ref[q, :, :] = gp[:, q * DQ:(q + 1) * DQ]
    sc_ref[...] = dinv[:, None] * gp + b_ref[0, :][None, :]
    dinv_ref[0, :] = dinv


def _tc_step_body(acc_ref, sc_in_ref, dinv_ref, w_ref, b_ref,
                  gpc_ref, sc_ref, o_ref):
    dinv = dinv_ref[0, :]
    acc = jnp.concatenate([acc_ref[q] for q in range(4)], axis=1)
    o = dinv[:, None] * acc + sc_in_ref[...]
    h = jnp.maximum(o, 0.0)
    hw = jnp.dot(h, w_ref[...], preferred_element_type=jnp.float32)
    gp = dinv[:, None] * hw
    for q in range(4):
        gpc_ref[q, :, :] = gp[:, q * DQ:(q + 1) * DQ]
    sc_ref[...] = dinv[:, None] * gp + b_ref[0, :][None, :]
    o_ref[...] = o


def _row_spec(shape):
    nd = len(shape)
    if nd == 2:
        return pl.BlockSpec((BLK, shape[1]), lambda i: (i, 0))
    return pl.BlockSpec((shape[0], BLK, shape[2]), lambda i: (0, i, 0))


_GRID = NP // BLK


def _tc1(xp, W1, b1r, p0, p1):
    return pl.pallas_call(
        _tc1_body,
        grid=(_GRID,),
        in_specs=[
            _row_spec(xp.shape),
            pl.BlockSpec(W1.shape, lambda i: (0, 0)),
            pl.BlockSpec(b1r.shape, lambda i: (0, 0)),
            pl.BlockSpec((1, BLK), lambda i: (0, i)),
            pl.BlockSpec((1, BLK), lambda i: (0, i)),
        ],
        out_specs=[
            pl.BlockSpec((4, BLK, DQ), lambda i: (0, i, 0)),
            pl.BlockSpec((BLK, 2 * D), lambda i: (i, 0)),
            pl.BlockSpec((1, BLK), lambda i: (0, i)),
        ],
        out_shape=[
            jax.ShapeDtypeStruct((4, NP, DQ), jnp.float32),
            jax.ShapeDtypeStruct((NP, 2 * D), jnp.float32),
            jax.ShapeDtypeStruct((1, NP), jnp.float32),
        ],
    )(xp, W1, b1r, p0, p1)


def _tc_step(acc, sc_in, dinv, W, br):
    return pl.pallas_call(
        _tc_step_body,
        grid=(_GRID,),
        in_specs=[
            _row_spec(acc.shape),
            _row_spec(sc_in.shape),
            pl.BlockSpec((1, BLK), lambda i: (0, i)),
            pl.BlockSpec(W.shape, lambda i: (0, 0)),
            pl.BlockSpec(br.shape, lambda i: (0, 0)),
        ],
        out_specs=[
            pl.BlockSpec((4, BLK, DQ), lambda i: (0, i, 0)),
            pl.BlockSpec((BLK, 2 * D), lambda i: (i, 0)),
            pl.BlockSpec((BLK, 2 * D), lambda i: (i, 0)),
        ],
        out_shape=[
            jax.ShapeDtypeStruct((4, NP, DQ), jnp.float32),
            jax.ShapeDtypeStruct((NP, 2 * D), jnp.float32),
            jax.ShapeDtypeStruct((NP, 2 * D), jnp.float32),
        ],
    )(acc, sc_in, dinv, W, br)


def kernel(x, edge_index, W1, b1, Wmu, bmu, Wls, bls):
    src = edge_index[0].astype(jnp.int32)
    dst = edge_index[1].astype(jnp.int32)
    pad = EP - E
    src_p = jnp.concatenate([src, jnp.zeros((pad,), jnp.int32)])
    dst_p = jnp.concatenate([dst, jnp.full((pad,), TRASH, jnp.int32)])
    src4 = jnp.stack([src_p + q * NP for q in range(4)]).reshape(4, NBT, 128)
    dst2d = dst_p.reshape(NBT, 128)
    xp = jnp.pad(x, ((0, NP - N), (0, 0)))
    W23 = jnp.concatenate([Wmu, Wls], axis=1)
    b23 = jnp.concatenate([bmu, bls]).reshape(1, 2 * D)
    b1r = b1.reshape(1, 2 * D)
    zrows = jnp.zeros((ROWS_PER_W, DQ), jnp.float32)
    zrow1 = jnp.zeros((NP,), jnp.float32)
    ones128 = jnp.ones((128,), jnp.float32)

    degp = _sc_degree(dst2d, ones128, zrow1)
    p0 = degp[:NP].reshape(1, NP)
    p1 = degp[NP:].reshape(1, NP)

    gpc1, sc1, dinv = _tc1(xp, W1, b1r, p0, p1)

    Wstack = jnp.stack([W23, jnp.zeros_like(W23)])
    bstack = jnp.stack([b23, jnp.zeros_like(b23)])

    def step(carry, wb):
        gpc, sc_in = carry
        W, br = wb
        acc = _sc_pass(gpc.reshape(4 * NP, DQ), src4, dst2d, zrows)
        gpc_n, sc_n, o = _tc_step(acc, sc_in, dinv, W, br)
        return (gpc_n, sc_n), o

    (_, _), os = lax.scan(step, (gpc1, sc1), (Wstack, bstack))
    out = os[1]
    return out[:N, :D], out[:N, D:]


# speedup vs baseline: 1.0815x; 1.0815x over previous
import functools

import jax
import jax.numpy as jnp
from jax import lax
from jax.experimental import pallas as pl
from jax.experimental.pallas import tpu as pltpu
from jax.experimental.pallas import tpu_sc as plsc

N = 10000
E = 320000
D = 128
NP = 10240
TRASH = 10200
EP = 327680
NBT = EP // 128
ROWS_PER_W = NP // 16
BLK = 1024

_mesh = plsc.VectorSubcoreMesh(core_axis_name="c", subcore_axis_name="s")


@functools.partial(
    pl.kernel,
    mesh=_mesh,
    out_type=jax.ShapeDtypeStruct((2 * NP,), jnp.float32),
    scratch_types=[
        pltpu.VMEM((NBT // 32, 128), jnp.int32),
        pltpu.VMEM((128,), jnp.float32),
        pltpu.VMEM((ROWS_PER_W,), jnp.float32),
        pltpu.VMEM_SHARED((2 * NP,), jnp.float32),
    ],
)
def _sc_degree(dst_hbm, ones_hbm, zrow_hbm, out_hbm, dst_v, ones_v, zv, deg_sp):
    c = lax.axis_index("c")
    s = lax.axis_index("s")
    w = c * 16 + s
    nb = NBT // 32
    pltpu.sync_copy(dst_hbm.at[pl.ds(w * nb, nb)], dst_v)
    pltpu.sync_copy(ones_hbm, ones_v)
    pltpu.sync_copy(zrow_hbm.at[pl.ds(0, ROWS_PER_W)], zv)
    pltpu.sync_copy(zv, deg_sp.at[pl.ds(s * ROWS_PER_W, ROWS_PER_W)])
    plsc.subcore_barrier()

    def body(i, carry):
        pltpu.sync_copy(ones_v, deg_sp.at[dst_v.at[i]], add=True)
        return carry

    lax.fori_loop(0, nb, body, 0)
    plsc.subcore_barrier()
    pltpu.sync_copy(deg_sp.at[pl.ds(s * ROWS_PER_W, ROWS_PER_W)], zv)
    pltpu.sync_copy(zv, out_hbm.at[pl.ds(c * NP + s * ROWS_PER_W, ROWS_PER_W)])


DQ = D // 2


@functools.partial(
    pl.kernel,
    mesh=_mesh,
    out_type=jax.ShapeDtypeStruct((4, NP, DQ), jnp.float32),
    scratch_types=[
        pltpu.VMEM((32, 128), jnp.int32),
        pltpu.VMEM((32, 128), jnp.int32),
        [pltpu.VMEM((128, DQ), jnp.float32) for _ in range(4)],
        pltpu.VMEM_SHARED((2 * NP, DQ), jnp.float32),
        [pltpu.SemaphoreType.DMA for _ in range(4)],
        [pltpu.SemaphoreType.DMA for _ in range(4)],
    ],
    compiler_params=pltpu.CompilerParams(use_tc_tiling_on_sc=False),
)
def _sc_pass(gpc_hbm, src4_hbm, dst_hbm, zrows_hbm, out_hbm,
             src_v, dst_v, rows, acc_sp, gsem, ssem):
    c = lax.axis_index("c")
    s = lax.axis_index("s")
    nb = NBT // 16
    nc = 32
    NCHUNK = nb // nc

    def gather(i, k):
        pltpu.async_copy(gpc_hbm.at[src_v.at[i]], rows[k], gsem[k])

    def wait_gather(i, k):
        pltpu.make_async_copy(gpc_hbm.at[src_v.at[i]], rows[k],
                              gsem[k]).wait()

    def scat(i, k):
        pltpu.async_copy(rows[k], acc_sp.at[dst_v.at[i]], ssem[k], add=True)

    def wait_scat(i, k):
        pltpu.make_async_copy(rows[k], acc_sp.at[dst_v.at[i]],
                              ssem[k]).wait()

    def body(j, carry):
        for k in range(4):
            b = 4 * j + k
            wait_gather(b, k)
            scat(b, k)
            m = b + 2
            km = (k + 2) % 4

            @pl.when(m < nc)
            def _():
                @pl.when(m >= 4)
                def _():
                    wait_scat(m - 4, km)

                gather(m, km)

        return carry

    for f in range(2):
        q = 2 * c + f
        pltpu.sync_copy(zrows_hbm,
                        acc_sp.at[pl.ds(s * ROWS_PER_W, ROWS_PER_W)])
        plsc.subcore_barrier()
        for h in range(NCHUNK):
            pltpu.sync_copy(src4_hbm.at[q, pl.ds(s * nb + h * nc, nc)], src_v)
            pltpu.sync_copy(dst_hbm.at[pl.ds(s * nb + h * nc, nc)], dst_v)
            gather(0, 0)
            gather(1, 1)
            lax.fori_loop(0, nc // 4, body, 0)
            for k in range(4):
                wait_scat(nc - 4 + k, k)
        plsc.subcore_barrier()
        pltpu.sync_copy(acc_sp.at[pl.ds(s * ROWS_PER_W, ROWS_PER_W)],
                        out_hbm.at[q, pl.ds(s * ROWS_PER_W, ROWS_PER_W)])


def _tc1_body(xp_ref, w_ref, b_ref, p0_ref, p1_ref,
              gpc_ref, sc_ref, dinv_ref):
    deg = 1.0 + p0_ref[0, :] + p1_ref[0, :]
    dinv = lax.rsqrt(deg)
    hw = jnp.dot(xp_ref[...], w_ref[...], preferred_element_type=jnp.float32)
    gp = dinv[:, None] * hw
    for q in range(4):
        gpc_ref[q, :, :] = gp[:, q * DQ:(q + 1) * DQ]
    sc_ref[...] = dinv[:, None] * gp + b_ref[0, :][None, :]
    dinv_ref[0, :] = dinv


def _tc_step_body(acc_ref, sc_in_ref, dinv_ref, w_ref, b_ref,
                  gpc_ref, sc_ref, o_ref):
    dinv = dinv_ref[0, :]
    acc = jnp.concatenate([acc_ref[q] for q in range(4)], axis=1)
    o = dinv[:, None] * acc + sc_in_ref[...]
    h = jnp.maximum(o, 0.0)
    hw = jnp.dot(h, w_ref[...], preferred_element_type=jnp.float32)
    gp = dinv[:, None] * hw
    for q in range(4):
        gpc_ref[q, :, :] = gp[:, q * DQ:(q + 1) * DQ]
    sc_ref[...] = dinv[:, None] * gp + b_ref[0, :][None, :]
    o_ref[...] = o


def _row_spec(shape):
    nd = len(shape)
    if nd == 2:
        return pl.BlockSpec((BLK, shape[1]), lambda i: (i, 0))
    return pl.BlockSpec((shape[0], BLK, shape[2]), lambda i: (0, i, 0))


_GRID = NP // BLK


def _tc1(xp, W1, b1r, p0, p1):
    return pl.pallas_call(
        _tc1_body,
        grid=(_GRID,),
        in_specs=[
            _row_spec(xp.shape),
            pl.BlockSpec(W1.shape, lambda i: (0, 0)),
            pl.BlockSpec(b1r.shape, lambda i: (0, 0)),
            pl.BlockSpec((1, BLK), lambda i: (0, i)),
            pl.BlockSpec((1, BLK), lambda i: (0, i)),
        ],
        out_specs=[
            pl.BlockSpec((4, BLK, DQ), lambda i: (0, i, 0)),
            pl.BlockSpec((BLK, 2 * D), lambda i: (i, 0)),
            pl.BlockSpec((1, BLK), lambda i: (0, i)),
        ],
        out_shape=[
            jax.ShapeDtypeStruct((4, NP, DQ), jnp.float32),
            jax.ShapeDtypeStruct((NP, 2 * D), jnp.float32),
            jax.ShapeDtypeStruct((1, NP), jnp.float32),
        ],
    )(xp, W1, b1r, p0, p1)


def _tc_step(acc, sc_in, dinv, W, br):
    return pl.pallas_call(
        _tc_step_body,
        grid=(_GRID,),
        in_specs=[
            _row_spec(acc.shape),
            _row_spec(sc_in.shape),
            pl.BlockSpec((1, BLK), lambda i: (0, i)),
            pl.BlockSpec(W.shape, lambda i: (0, 0)),
            pl.BlockSpec(br.shape, lambda i: (0, 0)),
        ],
        out_specs=[
            pl.BlockSpec((4, BLK, DQ), lambda i: (0, i, 0)),
            pl.BlockSpec((BLK, 2 * D), lambda i: (i, 0)),
            pl.BlockSpec((BLK, 2 * D), lambda i: (i, 0)),
        ],
        out_shape=[
            jax.ShapeDtypeStruct((4, NP, DQ), jnp.float32),
            jax.ShapeDtypeStruct((NP, 2 * D), jnp.float32),
            jax.ShapeDtypeStruct((NP, 2 * D), jnp.float32),
        ],
    )(acc, sc_in, dinv, W, br)


def kernel(x, edge_index, W1, b1, Wmu, bmu, Wls, bls):
    src = edge_index[0].astype(jnp.int32)
    dst = edge_index[1].astype(jnp.int32)
    pad = EP - E
    src_p = jnp.concatenate([src, jnp.zeros((pad,), jnp.int32)])
    dst_p = jnp.concatenate([dst, jnp.full((pad,), TRASH, jnp.int32)])
    src4 = jnp.stack([src_p + q * NP for q in range(4)]).reshape(4, NBT, 128)
    dst2d = dst_p.reshape(NBT, 128)
    xp = jnp.pad(x, ((0, NP - N), (0, 0)))
    W23 = jnp.concatenate([Wmu, Wls], axis=1)
    b23 = jnp.concatenate([bmu, bls]).reshape(1, 2 * D)
    b1r = b1.reshape(1, 2 * D)
    zrows = jnp.zeros((ROWS_PER_W, DQ), jnp.float32)
    zrow1 = jnp.zeros((NP,), jnp.float32)
    ones128 = jnp.ones((128,), jnp.float32)

    degp = _sc_degree(dst2d, ones128, zrow1)
    p0 = degp[:NP].reshape(1, NP)
    p1 = degp[NP:].reshape(1, NP)

    gpc1, sc1, dinv = _tc1(xp, W1, b1r, p0, p1)

    Wstack = jnp.stack([W23, jnp.zeros_like(W23)])
    bstack = jnp.stack([b23, jnp.zeros_like(b23)])

    def step(carry, wb):
        gpc, sc_in = carry
        W, br = wb
        acc = _sc_pass(gpc.reshape(4 * NP, DQ), src4, dst2d, zrows)
        gpc_n, sc_n, o = _tc_step(acc, sc_in, dinv, W, br)
        return (gpc_n, sc_n), o

    (_, _), os = lax.scan(step, (gpc1, sc1), (Wstack, bstack))
    out = os[1]
    return out[:N, :D], out[:N, D:]


# speedup vs baseline: 1.0934x; 1.0110x over previous
import functools

import jax
import jax.numpy as jnp
from jax import lax
from jax.experimental import pallas as pl
from jax.experimental.pallas import tpu as pltpu
from jax.experimental.pallas import tpu_sc as plsc

N = 10000
E = 320000
D = 128
NP = 10240
TRASH = 10200
EP = 327680
NBT = EP // 128
ROWS_PER_W = NP // 16
BLK = 1024

_mesh = plsc.VectorSubcoreMesh(core_axis_name="c", subcore_axis_name="s")


@functools.partial(
    pl.kernel,
    mesh=_mesh,
    out_type=jax.ShapeDtypeStruct((2 * NP,), jnp.float32),
    scratch_types=[
        pltpu.VMEM((NBT // 32, 128), jnp.int32),
        pltpu.VMEM((128,), jnp.float32),
        pltpu.VMEM((ROWS_PER_W,), jnp.float32),
        pltpu.VMEM_SHARED((2 * NP,), jnp.float32),
    ],
)
def _sc_degree(dst_hbm, ones_hbm, zrow_hbm, out_hbm, dst_v, ones_v, zv, deg_sp):
    c = lax.axis_index("c")
    s = lax.axis_index("s")
    w = c * 16 + s
    nb = NBT // 32
    pltpu.sync_copy(dst_hbm.at[pl.ds(w * nb, nb)], dst_v)
    pltpu.sync_copy(ones_hbm, ones_v)
    pltpu.sync_copy(zrow_hbm.at[pl.ds(0, ROWS_PER_W)], zv)
    pltpu.sync_copy(zv, deg_sp.at[pl.ds(s * ROWS_PER_W, ROWS_PER_W)])
    plsc.subcore_barrier()

    def body(i, carry):
        pltpu.sync_copy(ones_v, deg_sp.at[dst_v.at[i]], add=True)
        return carry

    lax.fori_loop(0, nb, body, 0)
    plsc.subcore_barrier()
    pltpu.sync_copy(deg_sp.at[pl.ds(s * ROWS_PER_W, ROWS_PER_W)], zv)
    pltpu.sync_copy(zv, out_hbm.at[pl.ds(c * NP + s * ROWS_PER_W, ROWS_PER_W)])


DQ = D // 2


@functools.partial(
    pl.kernel,
    mesh=_mesh,
    out_type=jax.ShapeDtypeStruct((4, NP, DQ), jnp.float32),
    scratch_types=[
        pltpu.VMEM((NBT // 32, 128), jnp.int32),
        pltpu.VMEM((NBT // 32, 128), jnp.int32),
        pltpu.VMEM((128, DQ), jnp.float32),
        pltpu.VMEM((128, DQ), jnp.float32),
        pltpu.VMEM_SHARED((2 * NP, DQ), jnp.float32),
        pltpu.SemaphoreType.DMA,
        pltpu.SemaphoreType.DMA,
    ],
    compiler_params=pltpu.CompilerParams(use_tc_tiling_on_sc=False),
)
def _sc_pass(gpc_hbm, src4_hbm, dst_hbm, zrows_hbm, out_hbm,
             src_v, dst_v, rows_a, rows_b, acc_sp, sem_a, sem_b):
    c = lax.axis_index("c")
    s = lax.axis_index("s")
    nb = NBT // 16
    nh = nb // 2

    def gather(i, buf, sem):
        return pltpu.async_copy(gpc_hbm.at[src_v.at[i]], buf, sem)

    def scat(i, buf):
        pltpu.sync_copy(buf, acc_sp.at[dst_v.at[i]], add=True)

    def body(j, carry):
        b0 = 2 * j
        gather(b0 + 1, rows_b, sem_b)
        pltpu.make_async_copy(gpc_hbm.at[src_v.at[b0]], rows_a, sem_a).wait()
        scat(b0, rows_a)

        @pl.when(b0 + 2 < nh)
        def _():
            gather(b0 + 2, rows_a, sem_a)

        pltpu.make_async_copy(gpc_hbm.at[src_v.at[b0 + 1]], rows_b,
                              sem_b).wait()
        scat(b0 + 1, rows_b)
        return carry

    for f in range(2):
        q = 2 * c + f
        pltpu.sync_copy(zrows_hbm,
                        acc_sp.at[pl.ds(s * ROWS_PER_W, ROWS_PER_W)])
        plsc.subcore_barrier()
        for h in range(2):
            pltpu.sync_copy(src4_hbm.at[q, pl.ds(s * nb + h * nh, nh)], src_v)
            pltpu.sync_copy(dst_hbm.at[pl.ds(s * nb + h * nh, nh)], dst_v)
            gather(0, rows_a, sem_a)
            lax.fori_loop(0, nh // 2, body, 0)
        plsc.subcore_barrier()
        pltpu.sync_copy(acc_sp.at[pl.ds(s * ROWS_PER_W, ROWS_PER_W)],
                        out_hbm.at[q, pl.ds(s * ROWS_PER_W, ROWS_PER_W)])


def _tc1_body(xp_ref, w_ref, b_ref, p0_ref, p1_ref,
              gpc_ref, sc_ref, dinv_ref):
    deg = 1.0 + p0_ref[0, :] + p1_ref[0, :]
    dinv = lax.rsqrt(deg)
    hw = jnp.dot(xp_ref[...], w_ref[...], preferred_element_type=jnp.float32)
    gp = dinv[:, None] * hw
    for q in range(4):
        gpc_ref[q, :, :] = gp[:, q * DQ:(q + 1) * DQ]
    sc_ref[...] = dinv[:, None] * gp + b_ref[0, :][None, :]
    dinv_ref[0, :] = dinv


def _tc_step_body(acc_ref, sc_in_ref, dinv_ref, w_ref, b_ref,
                  gpc_ref, sc_ref, o_ref):
    dinv = dinv_ref[0, :]
    acc = jnp.concatenate([acc_ref[q] for q in range(4)], axis=1)
    o = dinv[:, None] * acc + sc_in_ref[...]
    h = jnp.maximum(o, 0.0)
    hw = jnp.dot(h, w_ref[...], preferred_element_type=jnp.float32)
    gp = dinv[:, None] * hw
    for q in range(4):
        gpc_ref[q, :, :] = gp[:, q * DQ:(q + 1) * DQ]
    sc_ref[...] = dinv[:, None] * gp + b_ref[0, :][None, :]
    o_ref[...] = o


def _row_spec(shape):
    nd = len(shape)
    if nd == 2:
        return pl.BlockSpec((BLK, shape[1]), lambda i: (i, 0))
    return pl.BlockSpec((shape[0], BLK, shape[2]), lambda i: (0, i, 0))


_GRID = NP // BLK


def _tc1(xp, W1, b1r, p0, p1):
    return pl.pallas_call(
        _tc1_body,
        grid=(_GRID,),
        in_specs=[
            _row_spec(xp.shape),
            pl.BlockSpec(W1.shape, lambda i: (0, 0)),
            pl.BlockSpec(b1r.shape, lambda i: (0, 0)),
            pl.BlockSpec((1, BLK), lambda i: (0, i)),
            pl.BlockSpec((1, BLK), lambda i: (0, i)),
        ],
        out_specs=[
            pl.BlockSpec((4, BLK, DQ), lambda i: (0, i, 0)),
            pl.BlockSpec((BLK, 2 * D), lambda i: (i, 0)),
            pl.BlockSpec((1, BLK), lambda i: (0, i)),
        ],
        out_shape=[
            jax.ShapeDtypeStruct((4, NP, DQ), jnp.float32),
            jax.ShapeDtypeStruct((NP, 2 * D), jnp.float32),
            jax.ShapeDtypeStruct((1, NP), jnp.float32),
        ],
    )(xp, W1, b1r, p0, p1)


def _tc_step(acc, sc_in, dinv, W, br):
    return pl.pallas_call(
        _tc_step_body,
        grid=(_GRID,),
        in_specs=[
            _row_spec(acc.shape),
            _row_spec(sc_in.shape),
            pl.BlockSpec((1, BLK), lambda i: (0, i)),
            pl.BlockSpec(W.shape, lambda i: (0, 0)),
            pl.BlockSpec(br.shape, lambda i: (0, 0)),
        ],
        out_specs=[
            pl.BlockSpec((4, BLK, DQ), lambda i: (0, i, 0)),
            pl.BlockSpec((BLK, 2 * D), lambda i: (i, 0)),
            pl.BlockSpec((BLK, 2 * D), lambda i: (i, 0)),
        ],
        out_shape=[
            jax.ShapeDtypeStruct((4, NP, DQ), jnp.float32),
            jax.ShapeDtypeStruct((NP, 2 * D), jnp.float32),
            jax.ShapeDtypeStruct((NP, 2 * D), jnp.float32),
        ],
    )(acc, sc_in, dinv, W, br)


def kernel(x, edge_index, W1, b1, Wmu, bmu, Wls, bls):
    src = edge_index[0].astype(jnp.int32)
    dst = edge_index[1].astype(jnp.int32)
    pad = EP - E
    src_p = jnp.concatenate([src, jnp.zeros((pad,), jnp.int32)])
    dst_p = jnp.concatenate([dst, jnp.full((pad,), TRASH, jnp.int32)])
    src4 = jnp.stack([src_p + q * NP for q in range(4)]).reshape(4, NBT, 128)
    dst2d = dst_p.reshape(NBT, 128)
    xp = jnp.pad(x, ((0, NP - N), (0, 0)))
    W23 = jnp.concatenate([Wmu, Wls], axis=1)
    b23 = jnp.concatenate([bmu, bls]).reshape(1, 2 * D)
    b1r = b1.reshape(1, 2 * D)
    zrows = jnp.zeros((ROWS_PER_W, DQ), jnp.float32)
    zrow1 = jnp.zeros((NP,), jnp.float32)
    ones128 = jnp.ones((128,), jnp.float32)

    degp = _sc_degree(dst2d, ones128, zrow1)
    p0 = degp[:NP].reshape(1, NP)
    p1 = degp[NP:].reshape(1, NP)

    gpc1, sc1, dinv = _tc1(xp, W1, b1r, p0, p1)

    Wstack = jnp.stack([W23, jnp.zeros_like(W23)])
    bstack = jnp.stack([b23, jnp.zeros_like(b23)])

    def step(carry, wb):
        gpc, sc_in = carry
        W, br = wb
        acc = _sc_pass(gpc.reshape(4 * NP, DQ), src4, dst2d, zrows)
        gpc_n, sc_n, o = _tc_step(acc, sc_in, dinv, W, br)
        return (gpc_n, sc_n), o

    (_, _), os = lax.scan(step, (gpc1, sc1), (Wstack, bstack))
    out = os[1]
    return out[:N, :D], out[:N, D:]
